# Initial kernel scaffold; baseline (speedup 1.0000x reference)
import jax
import jax.numpy as jnp
from jax.experimental import pallas as pl


def kernel(batch_walk, node_embed, context_embed):
    raise NotImplementedError("write your pallas kernel here")


# speedup vs baseline: 2.1631x; 2.1631x over previous
import functools

import numpy as np
import jax
import jax.numpy as jnp
from jax import lax
from jax.experimental import pallas as pl
from jax.experimental.pallas import tpu as pltpu
from jax.experimental.pallas import tpu_sc as plsc

NUM_NODES = 100000
EMB_DIM = 128
WALK_LENGTH = 40
WINDOW_SIZE = 5
NEG_SIZE = 5
BATCH = 1024

N_POS = 370
N_POS_PAD = 384
N_NEG = N_POS * NEG_SIZE
N_NEG_PAD = 2048
NEG_CHUNK = 128
N_TILES = 32
ROWS_PER_TILE = BATCH // N_TILES
TOTAL_POS = BATCH * N_POS


def _build_pair_tables():
    src, dst = [], []
    for i in range(WALK_LENGTH):
        for j in range(max(0, i - WINDOW_SIZE), i):
            src.append(j)
            dst.append(i)
        for j in range(i + 1, min(WALK_LENGTH, i + 1 + WINDOW_SIZE)):
            src.append(j)
            dst.append(i)
    src = np.asarray(src, dtype=np.int32)
    dst = np.asarray(dst, dtype=np.int32)
    psrc = np.zeros((N_POS_PAD,), np.int32)
    pdst = np.zeros((N_POS_PAD,), np.int32)
    psrc[:N_POS] = src
    pdst[:N_POS] = dst
    nsrc = np.zeros((N_NEG_PAD,), np.int32)
    nsrc[:N_NEG] = np.repeat(dst, NEG_SIZE)
    return psrc, pdst, nsrc


_PSRC_NP, _PDST_NP, _NSRC_NP = _build_pair_tables()


def _threefry2x32_core(key1, key2, x0, x1):
    def rotl(x, d):
        return ((x << np.uint32(d)) | (x >> np.uint32(32 - d))).astype(np.uint32)

    x = [x0.astype(np.uint32).copy(), x1.astype(np.uint32).copy()]
    rot_a = (13, 15, 26, 6)
    rot_b = (17, 29, 16, 24)
    ks = [np.uint32(key1), np.uint32(key2),
          np.uint32(key1) ^ np.uint32(key2) ^ np.uint32(0x1BD11BDA)]

    def rounds(x, rots):
        for r in rots:
            x[0] = (x[0] + x[1]).astype(np.uint32)
            x[1] = rotl(x[1], r)
            x[1] = x[0] ^ x[1]
        return x

    x[0] = (x[0] + ks[0]).astype(np.uint32)
    x[1] = (x[1] + ks[1]).astype(np.uint32)
    x = rounds(x, rot_a)
    x[0] = (x[0] + ks[1]).astype(np.uint32)
    x[1] = (x[1] + ks[2] + np.uint32(1)).astype(np.uint32)
    x = rounds(x, rot_b)
    x[0] = (x[0] + ks[2]).astype(np.uint32)
    x[1] = (x[1] + ks[0] + np.uint32(2)).astype(np.uint32)
    x = rounds(x, rot_a)
    x[0] = (x[0] + ks[0]).astype(np.uint32)
    x[1] = (x[1] + ks[1] + np.uint32(3)).astype(np.uint32)
    x = rounds(x, rot_b)
    x[0] = (x[0] + ks[1]).astype(np.uint32)
    x[1] = (x[1] + ks[2] + np.uint32(4)).astype(np.uint32)
    x = rounds(x, rot_a)
    x[0] = (x[0] + ks[2]).astype(np.uint32)
    x[1] = (x[1] + ks[0] + np.uint32(5)).astype(np.uint32)
    return x[0], x[1]


def _np_permutation_key42(x):
    key = (np.uint32(0), np.uint32(42))
    exponent = 3
    num_rounds = int(np.ceil(exponent * np.log(max(1, x.size))
                             / np.log(np.iinfo(np.uint32).max)))
    for _ in range(num_rounds):
        z = np.zeros(2, np.uint32)
        b1, b2 = _threefry2x32_core(key[0], key[1], z,
                                    np.arange(2, dtype=np.uint32))
        key, subkey = (b1[0], b2[0]), (b1[1], b2[1])
        zn = np.zeros(x.size, np.uint32)
        s1, s2 = _threefry2x32_core(subkey[0], subkey[1], zn,
                                    np.arange(x.size, dtype=np.uint32))
        bits = s1 ^ s2
        order = np.argsort(bits, kind="stable")
        x = x[order]
    return x


_TILED_NP = np.tile(np.arange(BATCH * WALK_LENGTH, dtype=np.int32),
                    NEG_SIZE * WINDOW_SIZE * 2)
_PERM_NP = _np_permutation_key42(_TILED_NP)[: BATCH * N_NEG]
_NEGG_NP = np.zeros((BATCH, N_NEG_PAD), np.int32)
_NEGG_NP[:, :N_NEG] = _PERM_NP.reshape(BATCH, N_NEG)


def _sc_body(walk_hbm, node_hbm, ctx_hbm, negg_hbm, psrc_hbm, pdst_hbm,
             nsrc_hbm, posd_hbm, negd_hbm,
             walk_v, negg_v, negw_v, nego_v, poso_v,
             psrc_v, pdst_v, nsrc_v, nb_v, cb_v, ctxr_v, sem):
    cid = lax.axis_index("c")
    sid = lax.axis_index("s")
    wid = sid * 2 + cid

    pltpu.sync_copy(walk_hbm, walk_v)
    pltpu.sync_copy(psrc_hbm, psrc_v)
    pltpu.sync_copy(pdst_hbm, pdst_v)
    pltpu.sync_copy(nsrc_hbm, nsrc_v)

    def dot16(src_ref, srows, dst_ref, drows):
        def dim_step(d, acc):
            colv = jnp.full((16,), d, jnp.int32)
            sv = plsc.load_gather(src_ref, [srows, colv])
            dv = plsc.load_gather(dst_ref, [drows, colv])
            return acc + sv * dv
        return lax.fori_loop(0, EMB_DIM, dim_step,
                             jnp.zeros((16,), jnp.float32), unroll=8)

    def do_row(i, carry):
        b = wid * ROWS_PER_TILE + i
        wrow = walk_v.at[pl.ds(b * WALK_LENGTH, WALK_LENGTH)]
        pltpu.async_copy(node_hbm.at[wrow], nb_v, sem).wait()
        pltpu.async_copy(ctx_hbm.at[wrow], cb_v, sem).wait()

        def pos_g(gi, c2):
            srows = psrc_v[pl.ds(gi * 16, 16)]
            drows = pdst_v[pl.ds(gi * 16, 16)]
            poso_v[pl.ds(gi * 16, 16)] = dot16(nb_v, srows, cb_v, drows)
            return c2
        lax.fori_loop(0, N_POS_PAD // 16, pos_g, 0)
        pltpu.sync_copy(poso_v, posd_hbm.at[b])

        pltpu.sync_copy(negg_hbm.at[b], negg_v)

        def w_g(j, c2):
            g16 = negg_v[pl.ds(j * 16, 16)]
            negw_v[pl.ds(j * 16, 16)] = plsc.load_gather(walk_v, [g16])
            return c2
        lax.fori_loop(0, N_NEG_PAD // 16, w_g, 0)

        def neg_chunk(ch, c2):
            idx = negw_v.at[pl.ds(ch * NEG_CHUNK, NEG_CHUNK)]
            pltpu.async_copy(ctx_hbm.at[idx], ctxr_v, sem).wait()

            def neg_g(gi, c3):
                off = ch * NEG_CHUNK + gi * 16
                srows = nsrc_v[pl.ds(off, 16)]
                drows = gi * 16 + lax.iota(jnp.int32, 16)
                nego_v[pl.ds(off, 16)] = dot16(nb_v, srows, ctxr_v, drows)
                return c3
            lax.fori_loop(0, NEG_CHUNK // 16, neg_g, 0)
            return c2
        lax.fori_loop(0, N_NEG_PAD // NEG_CHUNK, neg_chunk, 0)
        pltpu.sync_copy(nego_v, negd_hbm.at[b])
        return carry

    lax.fori_loop(0, ROWS_PER_TILE, do_row, 0)


def _sc_dots(walk_flat, node_embed, context_embed, negg, psrc, pdst, nsrc):
    mesh = plsc.VectorSubcoreMesh(core_axis_name="c", subcore_axis_name="s")
    f = pl.kernel(
        _sc_body,
        out_type=(
            jax.ShapeDtypeStruct((BATCH, N_POS_PAD), jnp.float32),
            jax.ShapeDtypeStruct((BATCH, N_NEG_PAD), jnp.float32),
        ),
        mesh=mesh,
        compiler_params=pltpu.CompilerParams(needs_layout_passes=False),
        scratch_types=[
            pltpu.VMEM((BATCH * WALK_LENGTH,), jnp.int32),
            pltpu.VMEM((N_NEG_PAD,), jnp.int32),
            pltpu.VMEM((N_NEG_PAD,), jnp.int32),
            pltpu.VMEM((N_NEG_PAD,), jnp.float32),
            pltpu.VMEM((N_POS_PAD,), jnp.float32),
            pltpu.VMEM((N_POS_PAD,), jnp.int32),
            pltpu.VMEM((N_POS_PAD,), jnp.int32),
            pltpu.VMEM((N_NEG_PAD,), jnp.int32),
            pltpu.VMEM((WALK_LENGTH, EMB_DIM), jnp.float32),
            pltpu.VMEM((WALK_LENGTH, EMB_DIM), jnp.float32),
            pltpu.VMEM((NEG_CHUNK, EMB_DIM), jnp.float32),
            pltpu.SemaphoreType.DMA,
        ],
    )
    return f(walk_flat, node_embed, context_embed, negg, psrc, pdst, nsrc)


def _tc_reduce_body(pos_ref, neg_ref, out_ref):
    i = pl.program_id(0)
    p = pos_ref[...]
    pm = lax.broadcasted_iota(jnp.int32, p.shape, 1) < N_POS
    pc = jnp.clip(p, -6.0, 6.0)
    pv = jnp.where(pm, jnp.log1p(jnp.exp(-pc)), 0.0)
    n = neg_ref[...]
    nm = lax.broadcasted_iota(jnp.int32, n.shape, 1) < N_NEG
    nc = jnp.clip(n, -6.0, 6.0)
    nv = jnp.where(nm, jnp.log1p(jnp.exp(nc)), 0.0)
    tot = (jnp.sum(pv) + jnp.sum(nv)) * (1.0 / TOTAL_POS)

    @pl.when(i == 0)
    def _():
        out_ref[0, 0] = tot

    @pl.when(i > 0)
    def _():
        out_ref[0, 0] = out_ref[0, 0] + tot


def _tc_reduce(pos_d, neg_d):
    nblk = 8
    rows = BATCH // nblk
    return pl.pallas_call(
        _tc_reduce_body,
        grid=(nblk,),
        in_specs=[
            pl.BlockSpec((rows, N_POS_PAD), lambda i: (i, 0)),
            pl.BlockSpec((rows, N_NEG_PAD), lambda i: (i, 0)),
        ],
        out_specs=pl.BlockSpec(memory_space=pltpu.SMEM),
        out_shape=jax.ShapeDtypeStruct((1, 1), jnp.float32),
    )(pos_d, neg_d)


@jax.jit
def kernel(batch_walk, node_embed, context_embed):
    walk_flat = batch_walk.reshape(-1)
    negg = jnp.asarray(_NEGG_NP)
    psrc = jnp.asarray(_PSRC_NP)
    pdst = jnp.asarray(_PDST_NP)
    nsrc = jnp.asarray(_NSRC_NP)
    pos_d, neg_d = _sc_dots(walk_flat, node_embed, context_embed,
                            negg, psrc, pdst, nsrc)
    out = _tc_reduce(pos_d, neg_d)
    return out[0, 0]


# speedup vs baseline: 2.3452x; 1.0841x over previous
import functools

import numpy as np
import jax
import jax.numpy as jnp
from jax import lax
from jax.experimental import pallas as pl
from jax.experimental.pallas import tpu as pltpu
from jax.experimental.pallas import tpu_sc as plsc

NUM_NODES = 100000
EMB_DIM = 128
WALK_LENGTH = 40
WINDOW_SIZE = 5
NEG_SIZE = 5
BATCH = 1024

N_POS = 370
N_POS_PAD = 384
N_NEG = N_POS * NEG_SIZE
N_NEG_PAD = 2048
NEG_CHUNK = 128
N_TILES = 32
ROWS_PER_TILE = BATCH // N_TILES
TOTAL_POS = BATCH * N_POS


def _build_pair_tables():
    src, dst = [], []
    for i in range(WALK_LENGTH):
        for j in range(max(0, i - WINDOW_SIZE), i):
            src.append(j)
            dst.append(i)
        for j in range(i + 1, min(WALK_LENGTH, i + 1 + WINDOW_SIZE)):
            src.append(j)
            dst.append(i)
    src = np.asarray(src, dtype=np.int32)
    dst = np.asarray(dst, dtype=np.int32)
    psrc = np.zeros((N_POS_PAD,), np.int32)
    pdst = np.zeros((N_POS_PAD,), np.int32)
    psrc[:N_POS] = src
    pdst[:N_POS] = dst
    nsrc = np.zeros((N_NEG_PAD,), np.int32)
    nsrc[:N_NEG] = np.repeat(dst, NEG_SIZE)
    return psrc, pdst, nsrc


_PSRC_NP, _PDST_NP, _NSRC_NP = _build_pair_tables()


def _threefry2x32_core(key1, key2, x0, x1):
    def rotl(x, d):
        return ((x << np.uint32(d)) | (x >> np.uint32(32 - d))).astype(np.uint32)

    x = [x0.astype(np.uint32).copy(), x1.astype(np.uint32).copy()]
    rot_a = (13, 15, 26, 6)
    rot_b = (17, 29, 16, 24)
    ks = [np.uint32(key1), np.uint32(key2),
          np.uint32(key1) ^ np.uint32(key2) ^ np.uint32(0x1BD11BDA)]

    def rounds(x, rots):
        for r in rots:
            x[0] = (x[0] + x[1]).astype(np.uint32)
            x[1] = rotl(x[1], r)
            x[1] = x[0] ^ x[1]
        return x

    x[0] = (x[0] + ks[0]).astype(np.uint32)
    x[1] = (x[1] + ks[1]).astype(np.uint32)
    x = rounds(x, rot_a)
    x[0] = (x[0] + ks[1]).astype(np.uint32)
    x[1] = (x[1] + ks[2] + np.uint32(1)).astype(np.uint32)
    x = rounds(x, rot_b)
    x[0] = (x[0] + ks[2]).astype(np.uint32)
    x[1] = (x[1] + ks[0] + np.uint32(2)).astype(np.uint32)
    x = rounds(x, rot_a)
    x[0] = (x[0] + ks[0]).astype(np.uint32)
    x[1] = (x[1] + ks[1] + np.uint32(3)).astype(np.uint32)
    x = rounds(x, rot_b)
    x[0] = (x[0] + ks[1]).astype(np.uint32)
    x[1] = (x[1] + ks[2] + np.uint32(4)).astype(np.uint32)
    x = rounds(x, rot_a)
    x[0] = (x[0] + ks[2]).astype(np.uint32)
    x[1] = (x[1] + ks[0] + np.uint32(5)).astype(np.uint32)
    return x[0], x[1]


def _np_permutation_key42(x):
    key = (np.uint32(0), np.uint32(42))
    exponent = 3
    num_rounds = int(np.ceil(exponent * np.log(max(1, x.size))
                             / np.log(np.iinfo(np.uint32).max)))
    for _ in range(num_rounds):
        z = np.zeros(2, np.uint32)
        b1, b2 = _threefry2x32_core(key[0], key[1], z,
                                    np.arange(2, dtype=np.uint32))
        key, subkey = (b1[0], b2[0]), (b1[1], b2[1])
        zn = np.zeros(x.size, np.uint32)
        s1, s2 = _threefry2x32_core(subkey[0], subkey[1], zn,
                                    np.arange(x.size, dtype=np.uint32))
        bits = s1 ^ s2
        order = np.argsort(bits, kind="stable")
        x = x[order]
    return x


_TILED_NP = np.tile(np.arange(BATCH * WALK_LENGTH, dtype=np.int32),
                    NEG_SIZE * WINDOW_SIZE * 2)
_PERM_NP = _np_permutation_key42(_TILED_NP)[: BATCH * N_NEG]
_NEGG_NP = np.zeros((BATCH, N_NEG_PAD), np.int32)
_NEGG_NP[:, :N_NEG] = _PERM_NP.reshape(BATCH, N_NEG)


def _sc_body(walk_hbm, node_hbm, ctx_hbm, negg_hbm, psrc_hbm, pdst_hbm,
             nsrc_hbm, posd_hbm, negd_hbm,
             walk_v, negg_v, negw_v, nego_v, poso_v,
             psrc_v, pdst_v, nsrc_v, nb_v, cb_v, ctxr_v, sem):
    cid = lax.axis_index("c")
    sid = lax.axis_index("s")
    wid = sid * 2 + cid

    pltpu.sync_copy(walk_hbm, walk_v)
    pltpu.sync_copy(psrc_hbm, psrc_v)
    pltpu.sync_copy(pdst_hbm, pdst_v)
    pltpu.sync_copy(nsrc_hbm, nsrc_v)

    def dot16(src_ref, srows, dst_ref, drows):
        zero = jnp.zeros((16,), jnp.float32)

        def dim_step(d, c):
            a0, a1, a2, a3 = c

            def ld(dd):
                colv = jnp.full((16,), dd, jnp.int32)
                return (plsc.load_gather(src_ref, [srows, colv]) *
                        plsc.load_gather(dst_ref, [drows, colv]))
            return (a0 + ld(d), a1 + ld(d + 1),
                    a2 + ld(d + 2), a3 + ld(d + 3))
        a0, a1, a2, a3 = plsc.parallel_loop(
            0, EMB_DIM, 4, unroll=4, carry=(zero, zero, zero, zero))(dim_step)
        return (a0 + a1) + (a2 + a3)

    def do_row(i, carry):
        b = wid * ROWS_PER_TILE + i
        wrow = walk_v.at[pl.ds(b * WALK_LENGTH, WALK_LENGTH)]
        pltpu.async_copy(node_hbm.at[wrow], nb_v, sem).wait()
        pltpu.async_copy(ctx_hbm.at[wrow], cb_v, sem).wait()

        @plsc.parallel_loop(0, N_POS_PAD // 16, 1)
        def pos_g(gi):
            srows = psrc_v[pl.ds(gi * 16, 16)]
            drows = pdst_v[pl.ds(gi * 16, 16)]
            poso_v[pl.ds(gi * 16, 16)] = dot16(nb_v, srows, cb_v, drows)
        pltpu.sync_copy(poso_v, posd_hbm.at[b])

        pltpu.sync_copy(negg_hbm.at[b], negg_v)

        @plsc.parallel_loop(0, N_NEG_PAD // 16, 1, unroll=4)
        def w_g(j):
            g16 = negg_v[pl.ds(j * 16, 16)]
            negw_v[pl.ds(j * 16, 16)] = plsc.load_gather(walk_v, [g16])

        def neg_chunk(ch, c2):
            idx = negw_v.at[pl.ds(ch * NEG_CHUNK, NEG_CHUNK)]
            pltpu.async_copy(ctx_hbm.at[idx], ctxr_v, sem).wait()

            @plsc.parallel_loop(0, NEG_CHUNK // 16, 1)
            def neg_g(gi):
                off = ch * NEG_CHUNK + gi * 16
                srows = nsrc_v[pl.ds(off, 16)]
                drows = gi * 16 + lax.iota(jnp.int32, 16)
                nego_v[pl.ds(off, 16)] = dot16(nb_v, srows, ctxr_v, drows)
            return c2
        lax.fori_loop(0, N_NEG_PAD // NEG_CHUNK, neg_chunk, 0)
        pltpu.sync_copy(nego_v, negd_hbm.at[b])
        return carry

    lax.fori_loop(0, ROWS_PER_TILE, do_row, 0)


def _sc_dots(walk_flat, node_embed, context_embed, negg, psrc, pdst, nsrc):
    mesh = plsc.VectorSubcoreMesh(core_axis_name="c", subcore_axis_name="s")
    f = pl.kernel(
        _sc_body,
        out_type=(
            jax.ShapeDtypeStruct((BATCH, N_POS_PAD), jnp.float32),
            jax.ShapeDtypeStruct((BATCH, N_NEG_PAD), jnp.float32),
        ),
        mesh=mesh,
        compiler_params=pltpu.CompilerParams(needs_layout_passes=False),
        scratch_types=[
            pltpu.VMEM((BATCH * WALK_LENGTH,), jnp.int32),
            pltpu.VMEM((N_NEG_PAD,), jnp.int32),
            pltpu.VMEM((N_NEG_PAD,), jnp.int32),
            pltpu.VMEM((N_NEG_PAD,), jnp.float32),
            pltpu.VMEM((N_POS_PAD,), jnp.float32),
            pltpu.VMEM((N_POS_PAD,), jnp.int32),
            pltpu.VMEM((N_POS_PAD,), jnp.int32),
            pltpu.VMEM((N_NEG_PAD,), jnp.int32),
            pltpu.VMEM((WALK_LENGTH, EMB_DIM), jnp.float32),
            pltpu.VMEM((WALK_LENGTH, EMB_DIM), jnp.float32),
            pltpu.VMEM((NEG_CHUNK, EMB_DIM), jnp.float32),
            pltpu.SemaphoreType.DMA,
        ],
    )
    return f(walk_flat, node_embed, context_embed, negg, psrc, pdst, nsrc)


def _tc_reduce_body(pos_ref, neg_ref, out_ref):
    i = pl.program_id(0)
    p = pos_ref[...]
    pm = lax.broadcasted_iota(jnp.int32, p.shape, 1) < N_POS
    pc = jnp.clip(p, -6.0, 6.0)
    pv = jnp.where(pm, jnp.log1p(jnp.exp(-pc)), 0.0)
    n = neg_ref[...]
    nm = lax.broadcasted_iota(jnp.int32, n.shape, 1) < N_NEG
    nc = jnp.clip(n, -6.0, 6.0)
    nv = jnp.where(nm, jnp.log1p(jnp.exp(nc)), 0.0)
    tot = (jnp.sum(pv) + jnp.sum(nv)) * (1.0 / TOTAL_POS)

    @pl.when(i == 0)
    def _():
        out_ref[0, 0] = tot

    @pl.when(i > 0)
    def _():
        out_ref[0, 0] = out_ref[0, 0] + tot


def _tc_reduce(pos_d, neg_d):
    nblk = 8
    rows = BATCH // nblk
    return pl.pallas_call(
        _tc_reduce_body,
        grid=(nblk,),
        in_specs=[
            pl.BlockSpec((rows, N_POS_PAD), lambda i: (i, 0)),
            pl.BlockSpec((rows, N_NEG_PAD), lambda i: (i, 0)),
        ],
        out_specs=pl.BlockSpec(memory_space=pltpu.SMEM),
        out_shape=jax.ShapeDtypeStruct((1, 1), jnp.float32),
    )(pos_d, neg_d)


@jax.jit
def kernel(batch_walk, node_embed, context_embed):
    walk_flat = batch_walk.reshape(-1)
    negg = jnp.asarray(_NEGG_NP)
    psrc = jnp.asarray(_PSRC_NP)
    pdst = jnp.asarray(_PDST_NP)
    nsrc = jnp.asarray(_NSRC_NP)
    pos_d, neg_d = _sc_dots(walk_flat, node_embed, context_embed,
                            negg, psrc, pdst, nsrc)
    out = _tc_reduce(pos_d, neg_d)
    return out[0, 0]


# speedup vs baseline: 6.9266x; 2.9536x over previous
import functools

import numpy as np
import jax
import jax.numpy as jnp
from jax import lax
from jax.experimental import pallas as pl
from jax.experimental.pallas import tpu as pltpu
from jax.experimental.pallas import tpu_sc as plsc

NUM_NODES = 100000
EMB_DIM = 128
WALK_LENGTH = 40
WINDOW_SIZE = 5
NEG_SIZE = 5
BATCH = 1024

N_POS = 370
N_POS_PAD = 384
N_NEG = N_POS * NEG_SIZE
N_POSN_PAD = 384
N_NEG_PAD = N_POSN_PAD * NEG_SIZE
POS_PER_CHUNK = 16
NEG_CHUNK = POS_PER_CHUNK * NEG_SIZE
N_CHUNKS = N_POSN_PAD // POS_PER_CHUNK
N_TILES = 32
ROWS_PER_TILE = BATCH // N_TILES
TOTAL_POS = BATCH * N_POS


def _build_pair_tables():
    src, dst = [], []
    for i in range(WALK_LENGTH):
        for j in range(max(0, i - WINDOW_SIZE), i):
            src.append(j)
            dst.append(i)
        for j in range(i + 1, min(WALK_LENGTH, i + 1 + WINDOW_SIZE)):
            src.append(j)
            dst.append(i)
    src = np.asarray(src, dtype=np.int32)
    dst = np.asarray(dst, dtype=np.int32)
    psrc = np.zeros((N_POS_PAD,), np.int32)
    pdst = np.zeros((N_POS_PAD,), np.int32)
    psrc[:N_POS] = src
    pdst[:N_POS] = dst
    nsrcp = np.zeros((N_POSN_PAD,), np.int32)
    nsrcp[:N_POS] = dst
    return psrc, pdst, nsrcp


_PSRC_NP, _PDST_NP, _NSRCP_NP = _build_pair_tables()


def _threefry2x32_core(key1, key2, x0, x1):
    def rotl(x, d):
        return ((x << np.uint32(d)) | (x >> np.uint32(32 - d))).astype(np.uint32)

    x = [x0.astype(np.uint32).copy(), x1.astype(np.uint32).copy()]
    rot_a = (13, 15, 26, 6)
    rot_b = (17, 29, 16, 24)
    ks = [np.uint32(key1), np.uint32(key2),
          np.uint32(key1) ^ np.uint32(key2) ^ np.uint32(0x1BD11BDA)]

    def rounds(x, rots):
        for r in rots:
            x[0] = (x[0] + x[1]).astype(np.uint32)
            x[1] = rotl(x[1], r)
            x[1] = x[0] ^ x[1]
        return x

    x[0] = (x[0] + ks[0]).astype(np.uint32)
    x[1] = (x[1] + ks[1]).astype(np.uint32)
    x = rounds(x, rot_a)
    x[0] = (x[0] + ks[1]).astype(np.uint32)
    x[1] = (x[1] + ks[2] + np.uint32(1)).astype(np.uint32)
    x = rounds(x, rot_b)
    x[0] = (x[0] + ks[2]).astype(np.uint32)
    x[1] = (x[1] + ks[0] + np.uint32(2)).astype(np.uint32)
    x = rounds(x, rot_a)
    x[0] = (x[0] + ks[0]).astype(np.uint32)
    x[1] = (x[1] + ks[1] + np.uint32(3)).astype(np.uint32)
    x = rounds(x, rot_b)
    x[0] = (x[0] + ks[1]).astype(np.uint32)
    x[1] = (x[1] + ks[2] + np.uint32(4)).astype(np.uint32)
    x = rounds(x, rot_a)
    x[0] = (x[0] + ks[2]).astype(np.uint32)
    x[1] = (x[1] + ks[0] + np.uint32(5)).astype(np.uint32)
    return x[0], x[1]


def _np_permutation_key42(x):
    key = (np.uint32(0), np.uint32(42))
    exponent = 3
    num_rounds = int(np.ceil(exponent * np.log(max(1, x.size))
                             / np.log(np.iinfo(np.uint32).max)))
    for _ in range(num_rounds):
        z = np.zeros(2, np.uint32)
        b1, b2 = _threefry2x32_core(key[0], key[1], z,
                                    np.arange(2, dtype=np.uint32))
        key, subkey = (b1[0], b2[0]), (b1[1], b2[1])
        zn = np.zeros(x.size, np.uint32)
        s1, s2 = _threefry2x32_core(subkey[0], subkey[1], zn,
                                    np.arange(x.size, dtype=np.uint32))
        bits = s1 ^ s2
        order = np.argsort(bits, kind="stable")
        x = x[order]
    return x


_TILED_NP = np.tile(np.arange(BATCH * WALK_LENGTH, dtype=np.int32),
                    NEG_SIZE * WINDOW_SIZE * 2)
_PERM_NP = _np_permutation_key42(_TILED_NP)[: BATCH * N_NEG]
_NEGG_NP = np.zeros((BATCH, N_NEG_PAD), np.int32)
_NEGG_NP[:, :N_NEG] = _PERM_NP.reshape(BATCH, N_NEG)


def _sc_body(walk_hbm, node_hbm, ctx_hbm, negg_hbm, psrc_hbm, pdst_hbm,
             nsrcp_hbm, posd_hbm, negd_hbm,
             walk_v, negg_v, negw_v, nego_v, poso_v,
             psrc_v, pdst_v, nsrcp_v, nb_v, cb_v, ctxr_v, sem):
    cid = lax.axis_index("c")
    sid = lax.axis_index("s")
    wid = sid * 2 + cid

    pltpu.sync_copy(walk_hbm, walk_v)
    pltpu.sync_copy(psrc_hbm, psrc_v)
    pltpu.sync_copy(pdst_hbm, pdst_v)
    pltpu.sync_copy(nsrcp_hbm, nsrcp_v)

    lanes = lax.iota(jnp.int32, 16)

    def row_vecs(ref, r):
        return [ref[r, pl.ds(c * 16, 16)] for c in range(8)]

    def dot_vr(svecs, ref, r):
        a0 = svecs[0] * ref[r, pl.ds(0, 16)]
        a1 = svecs[1] * ref[r, pl.ds(16, 16)]
        for c in range(2, 8, 2):
            a0 = a0 + svecs[c] * ref[r, pl.ds(c * 16, 16)]
            a1 = a1 + svecs[c + 1] * ref[r, pl.ds((c + 1) * 16, 16)]
        return jnp.sum(a0 + a1)

    def do_row(i, carry):
        b = wid * ROWS_PER_TILE + i
        wrow = walk_v.at[pl.ds(b * WALK_LENGTH, WALK_LENGTH)]
        pltpu.async_copy(node_hbm.at[wrow], nb_v, sem).wait()
        pltpu.async_copy(ctx_hbm.at[wrow], cb_v, sem).wait()

        def pos_g(gi, c2):
            res = jnp.zeros((16,), jnp.float32)
            rs_vec = psrc_v[pl.ds(gi * 16, 16)]
            rd_vec = pdst_v[pl.ds(gi * 16, 16)]
            for k in range(16):
                rs = rs_vec[k]
                rd = rd_vec[k]
                svecs = row_vecs(nb_v, rs)
                res = jnp.where(lanes == k, dot_vr(svecs, cb_v, rd), res)
            poso_v[pl.ds(gi * 16, 16)] = res
            return c2
        lax.fori_loop(0, N_POS_PAD // 16, pos_g, 0)
        pltpu.sync_copy(poso_v, posd_hbm.at[b])

        pltpu.sync_copy(negg_hbm.at[b], negg_v)

        @plsc.parallel_loop(0, N_NEG_PAD // 16, 1, unroll=4)
        def w_g(j):
            g16 = negg_v[pl.ds(j * 16, 16)]
            negw_v[pl.ds(j * 16, 16)] = plsc.load_gather(walk_v, [g16])

        def neg_chunk(ch, c2):
            idx = negw_v.at[pl.ds(ch * NEG_CHUNK, NEG_CHUNK)]
            pltpu.async_copy(ctx_hbm.at[idx], ctxr_v, sem).wait()
            res = [jnp.zeros((16,), jnp.float32)] * NEG_SIZE
            rs_vec = nsrcp_v[pl.ds(ch * POS_PER_CHUNK, POS_PER_CHUNK)]
            for k in range(POS_PER_CHUNK):
                rs = rs_vec[k]
                svecs = row_vecs(nb_v, rs)
                for e in range(NEG_SIZE):
                    m = k * NEG_SIZE + e
                    s = dot_vr(svecs, ctxr_v, m)
                    res[m // 16] = jnp.where(lanes == (m % 16), s,
                                             res[m // 16])
            base = ch * NEG_CHUNK
            for v in range(NEG_SIZE):
                nego_v[pl.ds(base + v * 16, 16)] = res[v]
            return c2
        lax.fori_loop(0, N_CHUNKS, neg_chunk, 0)
        pltpu.sync_copy(nego_v, negd_hbm.at[b])
        return carry

    lax.fori_loop(0, ROWS_PER_TILE, do_row, 0)


def _sc_dots(walk_flat, node_embed, context_embed, negg, psrc, pdst, nsrc):
    mesh = plsc.VectorSubcoreMesh(core_axis_name="c", subcore_axis_name="s")
    f = pl.kernel(
        _sc_body,
        out_type=(
            jax.ShapeDtypeStruct((BATCH, N_POS_PAD), jnp.float32),
            jax.ShapeDtypeStruct((BATCH, N_NEG_PAD), jnp.float32),
        ),
        mesh=mesh,
        compiler_params=pltpu.CompilerParams(needs_layout_passes=False),
        scratch_types=[
            pltpu.VMEM((BATCH * WALK_LENGTH,), jnp.int32),
            pltpu.VMEM((N_NEG_PAD,), jnp.int32),
            pltpu.VMEM((N_NEG_PAD,), jnp.int32),
            pltpu.VMEM((N_NEG_PAD,), jnp.float32),
            pltpu.VMEM((N_POS_PAD,), jnp.float32),
            pltpu.VMEM((N_POS_PAD,), jnp.int32),
            pltpu.VMEM((N_POS_PAD,), jnp.int32),
            pltpu.VMEM((N_POSN_PAD,), jnp.int32),
            pltpu.VMEM((WALK_LENGTH, EMB_DIM), jnp.float32),
            pltpu.VMEM((WALK_LENGTH, EMB_DIM), jnp.float32),
            pltpu.VMEM((NEG_CHUNK, EMB_DIM), jnp.float32),
            pltpu.SemaphoreType.DMA,
        ],
    )
    return f(walk_flat, node_embed, context_embed, negg, psrc, pdst, nsrc)


def _tc_reduce_body(pos_ref, neg_ref, out_ref):
    i = pl.program_id(0)
    p = pos_ref[...]
    pm = lax.broadcasted_iota(jnp.int32, p.shape, 1) < N_POS
    pc = jnp.clip(p, -6.0, 6.0)
    pv = jnp.where(pm, jnp.log1p(jnp.exp(-pc)), 0.0)
    n = neg_ref[...]
    nm = lax.broadcasted_iota(jnp.int32, n.shape, 1) < N_NEG
    nc = jnp.clip(n, -6.0, 6.0)
    nv = jnp.where(nm, jnp.log1p(jnp.exp(nc)), 0.0)
    tot = (jnp.sum(pv) + jnp.sum(nv)) * (1.0 / TOTAL_POS)

    @pl.when(i == 0)
    def _():
        out_ref[0, 0] = tot

    @pl.when(i > 0)
    def _():
        out_ref[0, 0] = out_ref[0, 0] + tot


def _tc_reduce(pos_d, neg_d):
    nblk = 8
    rows = BATCH // nblk
    return pl.pallas_call(
        _tc_reduce_body,
        grid=(nblk,),
        in_specs=[
            pl.BlockSpec((rows, N_POS_PAD), lambda i: (i, 0)),
            pl.BlockSpec((rows, N_NEG_PAD), lambda i: (i, 0)),
        ],
        out_specs=pl.BlockSpec(memory_space=pltpu.SMEM),
        out_shape=jax.ShapeDtypeStruct((1, 1), jnp.float32),
    )(pos_d, neg_d)


@jax.jit
def kernel(batch_walk, node_embed, context_embed):
    walk_flat = batch_walk.reshape(-1)
    negg = jnp.asarray(_NEGG_NP)
    psrc = jnp.asarray(_PSRC_NP)
    pdst = jnp.asarray(_PDST_NP)
    nsrcp = jnp.asarray(_NSRCP_NP)
    pos_d, neg_d = _sc_dots(walk_flat, node_embed, context_embed,
                            negg, psrc, pdst, nsrcp)
    out = _tc_reduce(pos_d, neg_d)
    return out[0, 0]


# speedup vs baseline: 6.9563x; 1.0043x over previous
import functools

import numpy as np
import jax
import jax.numpy as jnp
from jax import lax
from jax.experimental import pallas as pl
from jax.experimental.pallas import tpu as pltpu
from jax.experimental.pallas import tpu_sc as plsc

NUM_NODES = 100000
EMB_DIM = 128
WALK_LENGTH = 40
WINDOW_SIZE = 5
NEG_SIZE = 5
BATCH = 1024

N_POS = 370
N_POS_PAD = 384
N_NEG = N_POS * NEG_SIZE
N_POSN_PAD = 384
N_NEG_PAD = N_POSN_PAD * NEG_SIZE
POS_PER_CHUNK = 16
NEG_CHUNK = POS_PER_CHUNK * NEG_SIZE
N_CHUNKS = N_POSN_PAD // POS_PER_CHUNK
N_TILES = 32
ROWS_PER_TILE = BATCH // N_TILES
TOTAL_POS = BATCH * N_POS


def _build_pair_tables():
    src, dst = [], []
    for i in range(WALK_LENGTH):
        for j in range(max(0, i - WINDOW_SIZE), i):
            src.append(j)
            dst.append(i)
        for j in range(i + 1, min(WALK_LENGTH, i + 1 + WINDOW_SIZE)):
            src.append(j)
            dst.append(i)
    src = np.asarray(src, dtype=np.int32)
    dst = np.asarray(dst, dtype=np.int32)
    psrc = np.zeros((N_POS_PAD,), np.int32)
    pdst = np.zeros((N_POS_PAD,), np.int32)
    psrc[:N_POS] = src
    pdst[:N_POS] = dst
    nsrcp = np.zeros((N_POSN_PAD,), np.int32)
    nsrcp[:N_POS] = dst
    return psrc, pdst, nsrcp


_PSRC_NP, _PDST_NP, _NSRCP_NP = _build_pair_tables()


def _threefry2x32_core(key1, key2, x0, x1):
    def rotl(x, d):
        return ((x << np.uint32(d)) | (x >> np.uint32(32 - d))).astype(np.uint32)

    x = [x0.astype(np.uint32).copy(), x1.astype(np.uint32).copy()]
    rot_a = (13, 15, 26, 6)
    rot_b = (17, 29, 16, 24)
    ks = [np.uint32(key1), np.uint32(key2),
          np.uint32(key1) ^ np.uint32(key2) ^ np.uint32(0x1BD11BDA)]

    def rounds(x, rots):
        for r in rots:
            x[0] = (x[0] + x[1]).astype(np.uint32)
            x[1] = rotl(x[1], r)
            x[1] = x[0] ^ x[1]
        return x

    x[0] = (x[0] + ks[0]).astype(np.uint32)
    x[1] = (x[1] + ks[1]).astype(np.uint32)
    x = rounds(x, rot_a)
    x[0] = (x[0] + ks[1]).astype(np.uint32)
    x[1] = (x[1] + ks[2] + np.uint32(1)).astype(np.uint32)
    x = rounds(x, rot_b)
    x[0] = (x[0] + ks[2]).astype(np.uint32)
    x[1] = (x[1] + ks[0] + np.uint32(2)).astype(np.uint32)
    x = rounds(x, rot_a)
    x[0] = (x[0] + ks[0]).astype(np.uint32)
    x[1] = (x[1] + ks[1] + np.uint32(3)).astype(np.uint32)
    x = rounds(x, rot_b)
    x[0] = (x[0] + ks[1]).astype(np.uint32)
    x[1] = (x[1] + ks[2] + np.uint32(4)).astype(np.uint32)
    x = rounds(x, rot_a)
    x[0] = (x[0] + ks[2]).astype(np.uint32)
    x[1] = (x[1] + ks[0] + np.uint32(5)).astype(np.uint32)
    return x[0], x[1]


def _np_permutation_key42(x):
    key = (np.uint32(0), np.uint32(42))
    exponent = 3
    num_rounds = int(np.ceil(exponent * np.log(max(1, x.size))
                             / np.log(np.iinfo(np.uint32).max)))
    for _ in range(num_rounds):
        z = np.zeros(2, np.uint32)
        b1, b2 = _threefry2x32_core(key[0], key[1], z,
                                    np.arange(2, dtype=np.uint32))
        key, subkey = (b1[0], b2[0]), (b1[1], b2[1])
        zn = np.zeros(x.size, np.uint32)
        s1, s2 = _threefry2x32_core(subkey[0], subkey[1], zn,
                                    np.arange(x.size, dtype=np.uint32))
        bits = s1 ^ s2
        order = np.argsort(bits, kind="stable")
        x = x[order]
    return x


_TILED_NP = np.tile(np.arange(BATCH * WALK_LENGTH, dtype=np.int32),
                    NEG_SIZE * WINDOW_SIZE * 2)
_PERM_NP = _np_permutation_key42(_TILED_NP)[: BATCH * N_NEG]
_NEGG_NP = np.zeros((BATCH, N_NEG_PAD), np.int32)
_NEGG_NP[:, :N_NEG] = _PERM_NP.reshape(BATCH, N_NEG)


def _sc_body(walk_hbm, node_hbm, ctx_hbm, negg_hbm, psrc_hbm, pdst_hbm,
             nsrcp_hbm, posd_hbm, negd_hbm,
             walk_v, negg_v, negw_v, nego_v, poso_v,
             psrc_v, pdst_v, nsrcp_v, nb_v, cb_v, ctxr_a, ctxr_b,
             sem, sem2, sem3, sem_a, sem_b):
    cid = lax.axis_index("c")
    sid = lax.axis_index("s")
    wid = sid * 2 + cid

    pltpu.sync_copy(walk_hbm, walk_v)
    pltpu.sync_copy(psrc_hbm, psrc_v)
    pltpu.sync_copy(pdst_hbm, pdst_v)
    pltpu.sync_copy(nsrcp_hbm, nsrcp_v)

    lanes = lax.iota(jnp.int32, 16)

    def row_vecs(ref, r):
        return [ref[r, pl.ds(c * 16, 16)] for c in range(8)]

    def dot_vr(svecs, ref, r):
        a0 = svecs[0] * ref[r, pl.ds(0, 16)]
        a1 = svecs[1] * ref[r, pl.ds(16, 16)]
        for c in range(2, 8, 2):
            a0 = a0 + svecs[c] * ref[r, pl.ds(c * 16, 16)]
            a1 = a1 + svecs[c + 1] * ref[r, pl.ds((c + 1) * 16, 16)]
        return jnp.sum(a0 + a1)

    def compute_chunk(ch, buf):
        res = [jnp.zeros((16,), jnp.float32)] * NEG_SIZE
        rs_vec = nsrcp_v[pl.ds(ch * POS_PER_CHUNK, POS_PER_CHUNK)]
        for k in range(POS_PER_CHUNK):
            rs = rs_vec[k]
            svecs = row_vecs(nb_v, rs)
            for e in range(NEG_SIZE):
                m = k * NEG_SIZE + e
                s = dot_vr(svecs, buf, m)
                res[m // 16] = jnp.where(lanes == (m % 16), s,
                                         res[m // 16])
        base = ch * NEG_CHUNK
        for v in range(NEG_SIZE):
            nego_v[pl.ds(base + v * 16, 16)] = res[v]

    def chunk_gather(ch, buf, csem):
        idx = negw_v.at[pl.ds(ch * NEG_CHUNK, NEG_CHUNK)]
        return pltpu.async_copy(ctx_hbm.at[idx], buf, csem)

    def chunk_wait(buf, csem):
        idx = negw_v.at[pl.ds(0, NEG_CHUNK)]
        pltpu.make_async_copy(ctx_hbm.at[idx], buf, csem).wait()

    def do_row(i, carry):
        b = wid * ROWS_PER_TILE + i
        wrow = walk_v.at[pl.ds(b * WALK_LENGTH, WALK_LENGTH)]
        cp_nb = pltpu.async_copy(node_hbm.at[wrow], nb_v, sem)
        cp_cb = pltpu.async_copy(ctx_hbm.at[wrow], cb_v, sem2)
        cp_gg = pltpu.async_copy(negg_hbm.at[b], negg_v, sem3)
        cp_gg.wait()

        @plsc.parallel_loop(0, N_NEG_PAD // 16, 1, unroll=4)
        def w_g(j):
            g16 = negg_v[pl.ds(j * 16, 16)]
            negw_v[pl.ds(j * 16, 16)] = plsc.load_gather(walk_v, [g16])

        cp_nb.wait()
        cp_cb.wait()
        chunk_gather(0, ctxr_a, sem_a)
        chunk_gather(1, ctxr_b, sem_b)

        def pos_g(gi, c2):
            res = jnp.zeros((16,), jnp.float32)
            rs_vec = psrc_v[pl.ds(gi * 16, 16)]
            rd_vec = pdst_v[pl.ds(gi * 16, 16)]
            for k in range(16):
                rs = rs_vec[k]
                rd = rd_vec[k]
                svecs = row_vecs(nb_v, rs)
                res = jnp.where(lanes == k, dot_vr(svecs, cb_v, rd), res)
            poso_v[pl.ds(gi * 16, 16)] = res
            return c2
        lax.fori_loop(0, N_POS_PAD // 16, pos_g, 0)
        pltpu.sync_copy(poso_v, posd_hbm.at[b])

        def chunk_pair(c2, c3):
            ch_a = 2 * c2
            chunk_wait(ctxr_a, sem_a)
            compute_chunk(ch_a, ctxr_a)

            @pl.when(c2 < N_CHUNKS // 2 - 1)
            def _():
                chunk_gather(ch_a + 2, ctxr_a, sem_a)
            chunk_wait(ctxr_b, sem_b)
            compute_chunk(ch_a + 1, ctxr_b)

            @pl.when(c2 < N_CHUNKS // 2 - 1)
            def _():
                chunk_gather(ch_a + 3, ctxr_b, sem_b)
            return c3
        lax.fori_loop(0, N_CHUNKS // 2, chunk_pair, 0)
        pltpu.sync_copy(nego_v, negd_hbm.at[b])
        return carry

    lax.fori_loop(0, ROWS_PER_TILE, do_row, 0)


def _sc_dots(walk_flat, node_embed, context_embed, negg, psrc, pdst, nsrc):
    mesh = plsc.VectorSubcoreMesh(core_axis_name="c", subcore_axis_name="s")
    f = pl.kernel(
        _sc_body,
        out_type=(
            jax.ShapeDtypeStruct((BATCH, N_POS_PAD), jnp.float32),
            jax.ShapeDtypeStruct((BATCH, N_NEG_PAD), jnp.float32),
        ),
        mesh=mesh,
        compiler_params=pltpu.CompilerParams(needs_layout_passes=False),
        scratch_types=[
            pltpu.VMEM((BATCH * WALK_LENGTH,), jnp.int32),
            pltpu.VMEM((N_NEG_PAD,), jnp.int32),
            pltpu.VMEM((N_NEG_PAD,), jnp.int32),
            pltpu.VMEM((N_NEG_PAD,), jnp.float32),
            pltpu.VMEM((N_POS_PAD,), jnp.float32),
            pltpu.VMEM((N_POS_PAD,), jnp.int32),
            pltpu.VMEM((N_POS_PAD,), jnp.int32),
            pltpu.VMEM((N_POSN_PAD,), jnp.int32),
            pltpu.VMEM((WALK_LENGTH, EMB_DIM), jnp.float32),
            pltpu.VMEM((WALK_LENGTH, EMB_DIM), jnp.float32),
            pltpu.VMEM((NEG_CHUNK, EMB_DIM), jnp.float32),
            pltpu.VMEM((NEG_CHUNK, EMB_DIM), jnp.float32),
            pltpu.SemaphoreType.DMA,
            pltpu.SemaphoreType.DMA,
            pltpu.SemaphoreType.DMA,
            pltpu.SemaphoreType.DMA,
            pltpu.SemaphoreType.DMA,
        ],
    )
    return f(walk_flat, node_embed, context_embed, negg, psrc, pdst, nsrc)


def _tc_reduce_body(pos_ref, neg_ref, out_ref):
    i = pl.program_id(0)
    p = pos_ref[...]
    pm = lax.broadcasted_iota(jnp.int32, p.shape, 1) < N_POS
    pc = jnp.clip(p, -6.0, 6.0)
    pv = jnp.where(pm, jnp.log1p(jnp.exp(-pc)), 0.0)
    n = neg_ref[...]
    nm = lax.broadcasted_iota(jnp.int32, n.shape, 1) < N_NEG
    nc = jnp.clip(n, -6.0, 6.0)
    nv = jnp.where(nm, jnp.log1p(jnp.exp(nc)), 0.0)
    tot = (jnp.sum(pv) + jnp.sum(nv)) * (1.0 / TOTAL_POS)

    @pl.when(i == 0)
    def _():
        out_ref[0, 0] = tot

    @pl.when(i > 0)
    def _():
        out_ref[0, 0] = out_ref[0, 0] + tot


def _tc_reduce(pos_d, neg_d):
    nblk = 8
    rows = BATCH // nblk
    return pl.pallas_call(
        _tc_reduce_body,
        grid=(nblk,),
        in_specs=[
            pl.BlockSpec((rows, N_POS_PAD), lambda i: (i, 0)),
            pl.BlockSpec((rows, N_NEG_PAD), lambda i: (i, 0)),
        ],
        out_specs=pl.BlockSpec(memory_space=pltpu.SMEM),
        out_shape=jax.ShapeDtypeStruct((1, 1), jnp.float32),
    )(pos_d, neg_d)


@jax.jit
def kernel(batch_walk, node_embed, context_embed):
    walk_flat = batch_walk.reshape(-1)
    negg = jnp.asarray(_NEGG_NP)
    psrc = jnp.asarray(_PSRC_NP)
    pdst = jnp.asarray(_PDST_NP)
    nsrcp = jnp.asarray(_NSRCP_NP)
    pos_d, neg_d = _sc_dots(walk_flat, node_embed, context_embed,
                            negg, psrc, pdst, nsrcp)
    out = _tc_reduce(pos_d, neg_d)
    return out[0, 0]
